# baseline (speedup 1.0000x reference)
import functools

import jax
import jax.numpy as jnp
from jax.experimental import pallas as pl
from jax.experimental.pallas import tpu as pltpu

_K_BLK = 3328


def _mm_body(xt_ref, wt_ref, o_ref, *, k_total, nk):
    k = pl.program_id(0)

    def contract(xb, wb):
        return jax.lax.dot_general(
            xb, wb, (((0,), (1,)), ((), ())),
            preferred_element_type=jnp.float32,
        )

    @pl.when(k == 0)
    def _first():
        o_ref[...] = contract(xt_ref[...], wt_ref[...])

    @pl.when(jnp.logical_and(k > 0, k < nk - 1))
    def _full():
        o_ref[...] += contract(xt_ref[...], wt_ref[...])

    @pl.when(k == nk - 1)
    def _tail():
        rem = k_total - (nk - 1) * _K_BLK
        o_ref[...] += contract(xt_ref[0:rem, :], wt_ref[:, 0:rem])


def kernel(x, W):
    m, k_total = x.shape
    _, n = W.shape
    nk = pl.cdiv(k_total, _K_BLK)
    xt = jnp.transpose(x)
    wt = jnp.transpose(W)
    return pl.pallas_call(
        functools.partial(_mm_body, k_total=k_total, nk=nk),
        grid=(nk,),
        in_specs=[
            pl.BlockSpec((_K_BLK, m), lambda k: (k, 0)),
            pl.BlockSpec((n, _K_BLK), lambda k: (0, k)),
        ],
        out_specs=pl.BlockSpec((m, n), lambda k: (0, 0)),
        out_shape=jax.ShapeDtypeStruct((m, n), jnp.float32),
        compiler_params=pltpu.CompilerParams(
            dimension_semantics=("arbitrary",),
        ),
    )(xt, wt)


# speedup vs baseline: 1.0059x; 1.0059x over previous
import functools

import jax
import jax.numpy as jnp
from jax.experimental import pallas as pl
from jax.experimental.pallas import tpu as pltpu

_K_BLK = 2816


def _mm_body(xt_ref, wt_ref, o_ref, *, k_total, nk):
    k = pl.program_id(0)

    def contract(xb, wb):
        return jax.lax.dot_general(
            xb, wb, (((0,), (1,)), ((), ())),
            preferred_element_type=jnp.float32,
        )

    @pl.when(k == 0)
    def _first():
        o_ref[...] = contract(xt_ref[...], wt_ref[...])

    @pl.when(jnp.logical_and(k > 0, k < nk - 1))
    def _full():
        o_ref[...] += contract(xt_ref[...], wt_ref[...])

    @pl.when(k == nk - 1)
    def _tail():
        rem = k_total - (nk - 1) * _K_BLK
        o_ref[...] += contract(xt_ref[0:rem, :], wt_ref[:, 0:rem])


def kernel(x, W):
    m, k_total = x.shape
    _, n = W.shape
    nk = pl.cdiv(k_total, _K_BLK)
    xt = jnp.transpose(x)
    wt = jnp.transpose(W)
    return pl.pallas_call(
        functools.partial(_mm_body, k_total=k_total, nk=nk),
        grid=(nk,),
        in_specs=[
            pl.BlockSpec((_K_BLK, m), lambda k: (k, 0)),
            pl.BlockSpec((n, _K_BLK), lambda k: (0, k)),
        ],
        out_specs=pl.BlockSpec((m, n), lambda k: (0, 0)),
        out_shape=jax.ShapeDtypeStruct((m, n), jnp.float32),
        compiler_params=pltpu.CompilerParams(
            dimension_semantics=("arbitrary",),
        ),
    )(xt, wt)


# speedup vs baseline: 1.0107x; 1.0048x over previous
import functools

import jax
import jax.numpy as jnp
from jax.experimental import pallas as pl
from jax.experimental.pallas import tpu as pltpu

_K_BLK = 3072


def _mm_body(xt_ref, wt_ref, o_ref, *, k_total, nk):
    k = pl.program_id(0)

    def contract(xb, wb):
        return jax.lax.dot_general(
            xb, wb, (((0,), (1,)), ((), ())),
            preferred_element_type=jnp.float32,
        )

    @pl.when(k == 0)
    def _first():
        o_ref[...] = contract(xt_ref[...], wt_ref[...])

    @pl.when(jnp.logical_and(k > 0, k < nk - 1))
    def _full():
        o_ref[...] += contract(xt_ref[...], wt_ref[...])

    @pl.when(k == nk - 1)
    def _tail():
        rem = k_total - (nk - 1) * _K_BLK
        o_ref[...] += contract(xt_ref[0:rem, :], wt_ref[:, 0:rem])


def kernel(x, W):
    m, k_total = x.shape
    _, n = W.shape
    nk = pl.cdiv(k_total, _K_BLK)
    xt = jnp.transpose(x)
    wt = jnp.transpose(W)
    return pl.pallas_call(
        functools.partial(_mm_body, k_total=k_total, nk=nk),
        grid=(nk,),
        in_specs=[
            pl.BlockSpec((_K_BLK, m), lambda k: (k, 0)),
            pl.BlockSpec((n, _K_BLK), lambda k: (0, k)),
        ],
        out_specs=pl.BlockSpec((m, n), lambda k: (0, 0)),
        out_shape=jax.ShapeDtypeStruct((m, n), jnp.float32),
        compiler_params=pltpu.CompilerParams(
            dimension_semantics=("arbitrary",),
        ),
    )(xt, wt)
